# Initial kernel scaffold; baseline (speedup 1.0000x reference)
import jax
import jax.numpy as jnp
from jax.experimental import pallas as pl


def kernel(text_embed, sentiment_ids, sentiment_table):
    raise NotImplementedError("write your pallas kernel here")


# speedup vs baseline: 1.7401x; 1.7401x over previous
import functools

import jax
import jax.numpy as jnp
from jax import lax
from jax.experimental import pallas as pl
from jax.experimental.pallas import tpu as pltpu
from jax.experimental.pallas import tpu_sc as plsc

B = 16384
TEXT_DIM = 256
SENT_DIM = 16
OUT_DIM = TEXT_DIM + SENT_DIM
L = 16

NUM_CORES = 2
NUM_SUBCORES = 16
NUM_WORKERS = NUM_CORES * NUM_SUBCORES
BPW = B // NUM_WORKERS
CHUNK = 128
NCHUNK = BPW // CHUNK


def _encode_body(text_hbm, ids_hbm, table_hbm, out_hbm,
                 idx_v, table_v, obuf0, obuf1,
                 rsem0, rsem1, wsem0, wsem1):
    wid = lax.axis_index("s") * NUM_CORES + lax.axis_index("c")
    base = wid * BPW

    pltpu.sync_copy(ids_hbm.at[pl.ds(base, BPW)], idx_v)
    pltpu.sync_copy(table_hbm, table_v)

    bufs = (obuf0, obuf1)
    rsems = (rsem0, rsem1)
    wsems = (wsem0, wsem1)
    in_cp = [None, None]
    out_cp = [None, None]
    lane = lax.iota(jnp.int32, L)

    in_cp[0] = pltpu.make_async_copy(
        text_hbm.at[pl.ds(base, CHUNK)], bufs[0].at[:, pl.ds(0, TEXT_DIM)],
        rsems[0])
    in_cp[0].start()

    for c in range(NCHUNK):
        b = c % 2
        nb = (c + 1) % 2
        if c + 1 < NCHUNK:
            if out_cp[nb] is not None:
                out_cp[nb].wait()
                out_cp[nb] = None
            in_cp[nb] = pltpu.make_async_copy(
                text_hbm.at[pl.ds(base + (c + 1) * CHUNK, CHUNK)],
                bufs[nb].at[:, pl.ds(0, TEXT_DIM)], rsems[nb])
            in_cp[nb].start()

        def lookup_group(p, _, _buf=bufs[b], _c=c):
            rows = p * L + lane
            ids_vec = idx_v[pl.ds(_c * CHUNK + p * L, L)]
            for j in range(SENT_DIM):
                col_j = jnp.full((L,), TEXT_DIM + j, jnp.int32)
                vals = plsc.load_gather(table_v, [ids_vec, col_j - TEXT_DIM])
                plsc.store_scatter(_buf, [rows, col_j], vals)
            return 0

        lax.fori_loop(0, CHUNK // L, lookup_group, 0)

        in_cp[b].wait()
        out_cp[b] = pltpu.make_async_copy(
            bufs[b], out_hbm.at[pl.ds(base + c * CHUNK, CHUNK)], wsems[b])
        out_cp[b].start()

    for b in range(2):
        if out_cp[b] is not None:
            out_cp[b].wait()


@functools.partial(jax.jit, static_argnames=())
def kernel(text_embed, sentiment_ids, sentiment_table):
    ids32 = sentiment_ids.astype(jnp.int32)
    mesh = plsc.VectorSubcoreMesh(core_axis_name="c", subcore_axis_name="s")
    enc = pl.kernel(
        _encode_body,
        mesh=mesh,
        compiler_params=pltpu.CompilerParams(needs_layout_passes=False),
        out_type=jax.ShapeDtypeStruct((B, OUT_DIM), jnp.float32),
        scratch_types=[
            pltpu.VMEM((BPW,), jnp.int32),
            pltpu.VMEM((3, SENT_DIM), jnp.float32),
            pltpu.VMEM((CHUNK, OUT_DIM), jnp.float32),
            pltpu.VMEM((CHUNK, OUT_DIM), jnp.float32),
            pltpu.SemaphoreType.DMA,
            pltpu.SemaphoreType.DMA,
            pltpu.SemaphoreType.DMA,
            pltpu.SemaphoreType.DMA,
        ],
    )
    return enc(text_embed, ids32, sentiment_table)
